# baseline (speedup 1.0000x reference)
import functools

import jax
import jax.numpy as jnp
import numpy as np
from jax import lax
from jax.experimental import pallas as pl
from jax.experimental.pallas import tpu as pltpu
from jax.experimental.pallas import tpu_sc as plsc

B = 128
T = 16
NSEG = 96
NL = 12
ST = 8
D = 1280
BB = 32
NK = ST * (NL - 1)


def _seg_sum_mat():
    m = np.zeros((8 * T, 8 * T), np.float32)
    for s in range(ST):
        m[s * T:(s + 1) * T, s * T:(s + 1) * T] = 1.0
    return m


def _fold_mat():
    m = np.zeros((8 * T, T), np.float32)
    for l in range(8 * T):
        m[l, l % T] = 1.0
    return m


_SEG_M = _seg_sum_mat()
_FOLD_M = _fold_mat()


def _tc_body(w_ref, integ_ref, g_ref, gf_ref, segm_ref,
             foldm_ref, mc_ref, idx_ref):
    w = w_ref[...]
    integ = integ_ref[...]
    g = g_ref[...]
    lw = jnp.log(w + 1e-10)
    lwt = jnp.concatenate([lw] * ST, axis=1)
    wt = jnp.concatenate([w] * ST, axis=1)

    p = lwt[:, None, :] + g
    lane = lax.broadcasted_iota(jnp.int32, (1, 1, 8 * T), 2)
    rank = jnp.zeros((BB, NL - 1, 8 * T), jnp.float32)
    for r in range(1, T):
        ra = pltpu.roll(p, 8 * T - r, 2)
        rb = pltpu.roll(p, T - r, 2)
        tmask = (lane % T) < (T - r)
        partner = jnp.where(tmask, ra, rb)
        rank = rank + (partner > p).astype(jnp.float32)
    ivec = lax.broadcasted_iota(jnp.int32, (BB, NL - 1, 8 * T), 1) + 1
    keep = (rank >= ivec.astype(jnp.float32)).astype(jnp.float32)
    nw = wt[:, None, :] * keep
    nw2 = nw.reshape(BB * (NL - 1), 8 * T)
    denom = jax.lax.dot(nw2, segm_ref[...],
                        precision=jax.lax.Precision.HIGHEST)
    contrib = (nw2 / denom).reshape(BB, NL - 1, 8 * T)
    csum = jnp.sum(contrib, axis=1)
    vsum = jax.lax.dot(csum, foldm_ref[...],
                       precision=jax.lax.Precision.HIGHEST)
    wsum = jnp.sum(w, axis=1, keepdims=True)
    v = (vsum + float(ST) * (w / wsum)) * (1.0 / float(ST * NL))

    gf2 = gf_ref[...].reshape(BB * T, D)
    vt = jnp.concatenate([v] * BB, axis=1)
    lane2 = lax.broadcasted_iota(jnp.int32, (BB, BB * T), 1)
    row2 = lax.broadcasted_iota(jnp.int32, (BB, BB * T), 0)
    vexp = jnp.where(lane2 // T == row2, vt, 0.0)
    mc_ref[...] = jax.lax.dot(vexp, gf2)

    jthr = (lax.broadcasted_iota(jnp.int32, (BB, NL), 1).astype(jnp.float32)
            + 0.5) / float(NL)
    intb = integ[:, None, :]
    thrb = jthr[:, :, None]
    lt = intb < thrb
    fc = jnp.sum(lt.astype(jnp.int32), axis=2) - 1
    y1 = jnp.max(jnp.where(lt, intb, -1.0), axis=2)
    y2 = jnp.min(jnp.where(lt, 3.0, intb), axis=2)
    x1 = fc.astype(jnp.float32) / float(T)
    x2 = (fc + 1).astype(jnp.float32) / float(T)
    quant = (x2 * (jthr - y1) - x1 * (jthr - y2)) / (y2 - y1)
    ints = jnp.floor(quant * float(NSEG)).astype(jnp.int32)

    cols = [jnp.maximum(ints[:, 0:1], 0)]
    for i in range(1, NL):
        prev = cols[-1]
        cur = ints[:, i:i + 1]
        cols.append(jnp.where(cur > prev, cur, prev + 1))
    intsm = jnp.concatenate(cols, axis=1)
    lim = (NSEG - NL) + lax.broadcasted_iota(jnp.int32, (BB, NL), 1)
    intsm = jnp.maximum(jnp.minimum(intsm, lim), 0)
    brow = pl.program_id(0) * BB + lax.broadcasted_iota(jnp.int32, (BB, NL), 0)
    idx_ref[...] = intsm + brow * NSEG


_NC = 2
_NS = 16
_NW = _NC * _NS
_BPW = (B * NL) // _NW
_MCW = B // _NW


def _sc_gather_body(table_hbm, idx_hbm, mc_hbm, out_hbm, idx_v, rows_v, mc_v,
                    sem):
    wid = lax.axis_index("s") * _NC + lax.axis_index("c")
    base = wid * _BPW
    pltpu.sync_copy(idx_hbm.at[pl.ds(base, _BPW)], idx_v)
    cp = pltpu.async_copy(table_hbm.at[idx_v], rows_v, sem)
    pltpu.sync_copy(mc_hbm.at[pl.ds(wid * _MCW, _MCW)], mc_v)
    pltpu.sync_copy(mc_v, out_hbm.at[pl.ds(wid * _MCW, _MCW)])
    cp.wait()
    pltpu.sync_copy(rows_v, out_hbm.at[pl.ds(B + base, _BPW)])


def _sc_gather(table, idx, mc):
    run = functools.partial(
        pl.kernel,
        mesh=plsc.VectorSubcoreMesh(
            core_axis_name="c", subcore_axis_name="s",
            num_cores=_NC, num_subcores=_NS),
        out_type=jax.ShapeDtypeStruct((B + B * NL, D), jnp.float32),
        scratch_types=[
            pltpu.VMEM((_BPW,), jnp.int32),
            pltpu.VMEM((_BPW, D), jnp.float32),
            pltpu.VMEM((_MCW, D), jnp.float32),
            pltpu.SemaphoreType.DMA,
        ],
    )(_sc_gather_body)
    return run(table, idx, mc)


def _threefry2x32(k1, k2, x1, x2):
    rotl = lambda x, d: (x << np.uint32(d)) | (x >> np.uint32(32 - d))
    ks = (k1, k2, k1 ^ k2 ^ np.uint32(0x1BD11BDA))
    rotations = ((13, 15, 26, 6), (17, 29, 16, 24))
    x1 = x1 + ks[0]
    x2 = x2 + ks[1]
    for i in range(5):
        for r in rotations[i % 2]:
            x1 = x1 + x2
            x2 = rotl(x2, r)
            x2 = x2 ^ x1
        x1 = x1 + ks[(i + 1) % 3]
        x2 = x2 + ks[(i + 2) % 3] + np.uint32(i + 1)
    return x1, x2


def _gumbel_tables():
    k1, k2 = np.uint32(0), np.uint32(42)
    gs = []
    for i in range(1, NL):
        f1, f2 = _threefry2x32(k1, k2, np.uint32(0), np.uint32(i))
        n = B * ST * T
        c = np.arange(n, dtype=np.uint32)
        o1, o2 = _threefry2x32(f1, f2, np.zeros(n, np.uint32), c)
        bits = o1 ^ o2
        u = ((bits >> np.uint32(9)) | np.uint32(0x3F800000)).view(np.float32)
        u = np.maximum(np.float32(0.0), u - np.float32(1.0))
        u = u.reshape(B * ST, T)
        gi = -np.log(-np.log(u + np.float32(1e-10)) + np.float32(1e-10))
        gs.append(gi.astype(np.float32).reshape(B, ST, T))
    return np.stack(gs, axis=1).reshape(B, NL - 1, ST * T)


_G_TABLE = _gumbel_tables()


def kernel(logits, global_feat, input_feat):
    w = jax.nn.softmax(logits, axis=1)
    cums = jnp.cumsum(w, axis=1)
    integ = jnp.concatenate([jnp.zeros((B, 1), w.dtype), cums], axis=1)
    integ = integ.at[:, -1].set(1.0)
    g = jnp.asarray(_G_TABLE)

    mc, idx = pl.pallas_call(
        _tc_body,
        grid=(B // BB,),
        in_specs=[
            pl.BlockSpec((BB, T), lambda i: (i, 0)),
            pl.BlockSpec((BB, T + 1), lambda i: (i, 0)),
            pl.BlockSpec((BB, NL - 1, ST * T), lambda i: (i, 0, 0)),
            pl.BlockSpec((BB, T, D), lambda i: (i, 0, 0)),
            pl.BlockSpec((ST * T, ST * T), lambda i: (0, 0)),
            pl.BlockSpec((ST * T, T), lambda i: (0, 0)),
        ],
        out_specs=[
            pl.BlockSpec((BB, D), lambda i: (i, 0)),
            pl.BlockSpec((BB, NL), lambda i: (i, 0)),
        ],
        out_shape=[
            jax.ShapeDtypeStruct((B, D), jnp.float32),
            jax.ShapeDtypeStruct((B, NL), jnp.int32),
        ],
    )(w, integ, g, global_feat,
      jnp.asarray(_SEG_M), jnp.asarray(_FOLD_M))

    sampled = jnp.take(input_feat, idx.reshape(-1), axis=0)
    return jnp.concatenate([mc, sampled], axis=0)


# speedup vs baseline: 1.1637x; 1.1637x over previous
import functools

import jax
import jax.numpy as jnp
import numpy as np
from jax import lax
from jax.experimental import pallas as pl
from jax.experimental.pallas import tpu as pltpu
from jax.experimental.pallas import tpu_sc as plsc

B = 128
T = 16
NSEG = 96
NL = 12
ST = 8
D = 1280
BB = 32
NK = ST * (NL - 1)


def _seg_sum_mat():
    m = np.zeros((8 * T, 8 * T), np.float32)
    for s in range(ST):
        m[s * T:(s + 1) * T, s * T:(s + 1) * T] = 1.0
    return m


def _fold_mat():
    m = np.zeros((8 * T, T), np.float32)
    for l in range(8 * T):
        m[l, l % T] = 1.0
    return m


_SEG_M = _seg_sum_mat()
_FOLD_M = _fold_mat()


def _tc_body(w_ref, cums_ref, g_ref, gf_ref, segm_ref,
             foldm_ref, mc_ref, idx_ref):
    w = w_ref[...]
    cums = cums_ref[...]
    g = g_ref[...]
    lw = jnp.log(w + 1e-10)
    lwt = jnp.concatenate([lw] * ST, axis=1)
    wt = jnp.concatenate([w] * ST, axis=1)

    p = lwt[:, None, :] + g
    lane = lax.broadcasted_iota(jnp.int32, (1, 1, 8 * T), 2)
    rank = jnp.zeros((BB, NL - 1, 8 * T), jnp.float32)
    for r in range(1, T):
        ra = pltpu.roll(p, 8 * T - r, 2)
        rb = pltpu.roll(p, T - r, 2)
        tmask = (lane % T) < (T - r)
        partner = jnp.where(tmask, ra, rb)
        rank = rank + (partner > p).astype(jnp.float32)
    ivec = lax.broadcasted_iota(jnp.int32, (BB, NL - 1, 8 * T), 1) + 1
    keep = (rank >= ivec.astype(jnp.float32)).astype(jnp.float32)
    nw = wt[:, None, :] * keep
    nw2 = nw.reshape(BB * (NL - 1), 8 * T)
    denom = jax.lax.dot(nw2, segm_ref[...],
                        precision=jax.lax.Precision.HIGHEST)
    contrib = (nw2 / denom).reshape(BB, NL - 1, 8 * T)
    csum = jnp.sum(contrib, axis=1)
    vsum = jax.lax.dot(csum, foldm_ref[...],
                       precision=jax.lax.Precision.HIGHEST)
    wsum = jnp.sum(w, axis=1, keepdims=True)
    v = (vsum + float(ST) * (w / wsum)) * (1.0 / float(ST * NL))

    gf2 = gf_ref[...].reshape(BB * T, D)
    vt = jnp.concatenate([v] * BB, axis=1)
    lane2 = lax.broadcasted_iota(jnp.int32, (BB, BB * T), 1)
    row2 = lax.broadcasted_iota(jnp.int32, (BB, BB * T), 0)
    vexp = jnp.where(lane2 // T == row2, vt, 0.0)
    mc_ref[...] = jax.lax.dot(vexp, gf2)

    jthr = (lax.broadcasted_iota(jnp.int32, (BB, NL), 1).astype(jnp.float32)
            + 0.5) / float(NL)
    cumsb = cums[:, None, :]
    thrb = jthr[:, :, None]
    laneT = lax.broadcasted_iota(jnp.int32, (1, 1, T), 2)
    valid = laneT < (T - 1)
    ltj = (cumsb < thrb) & valid
    fc = jnp.sum(ltj.astype(jnp.int32), axis=2)
    y1 = jnp.maximum(jnp.max(jnp.where(ltj, cumsb, -1.0), axis=2), 0.0)
    gej = jnp.logical_not(cumsb < thrb) & valid
    y2 = jnp.minimum(jnp.min(jnp.where(gej, cumsb, 3.0), axis=2), 1.0)
    x1 = fc.astype(jnp.float32) / float(T)
    x2 = (fc + 1).astype(jnp.float32) / float(T)
    quant = (x2 * (jthr - y1) - x1 * (jthr - y2)) / (y2 - y1)
    ints = jnp.floor(quant * float(NSEG)).astype(jnp.int32)

    cols = [jnp.maximum(ints[:, 0:1], 0)]
    for i in range(1, NL):
        prev = cols[-1]
        cur = ints[:, i:i + 1]
        cols.append(jnp.where(cur > prev, cur, prev + 1))
    intsm = jnp.concatenate(cols, axis=1)
    lim = (NSEG - NL) + lax.broadcasted_iota(jnp.int32, (BB, NL), 1)
    intsm = jnp.maximum(jnp.minimum(intsm, lim), 0)
    brow = pl.program_id(0) * BB + lax.broadcasted_iota(jnp.int32, (BB, NL), 0)
    idx_ref[...] = intsm + brow * NSEG


_NC = 2
_NS = 16
_NW = _NC * _NS
_BPW = (B * NL) // _NW
_MCW = B // _NW


def _sc_gather_body(table_hbm, idx_hbm, mc_hbm, out_hbm, idx_v, rows_v, mc_v,
                    sem):
    wid = lax.axis_index("s") * _NC + lax.axis_index("c")
    base = wid * _BPW
    pltpu.sync_copy(idx_hbm.at[pl.ds(base, _BPW)], idx_v)
    cp = pltpu.async_copy(table_hbm.at[idx_v], rows_v, sem)
    pltpu.sync_copy(mc_hbm.at[pl.ds(wid * _MCW, _MCW)], mc_v)
    pltpu.sync_copy(mc_v, out_hbm.at[pl.ds(wid * _MCW, _MCW)])
    cp.wait()
    pltpu.sync_copy(rows_v, out_hbm.at[pl.ds(B + base, _BPW)])


def _sc_gather(table, idx, mc):
    run = functools.partial(
        pl.kernel,
        mesh=plsc.VectorSubcoreMesh(
            core_axis_name="c", subcore_axis_name="s",
            num_cores=_NC, num_subcores=_NS),
        out_type=jax.ShapeDtypeStruct((B + B * NL, D), jnp.float32),
        scratch_types=[
            pltpu.VMEM((_BPW,), jnp.int32),
            pltpu.VMEM((_BPW, D), jnp.float32),
            pltpu.VMEM((_MCW, D), jnp.float32),
            pltpu.SemaphoreType.DMA,
        ],
    )(_sc_gather_body)
    return run(table, idx, mc)


def _threefry2x32(k1, k2, x1, x2):
    rotl = lambda x, d: (x << np.uint32(d)) | (x >> np.uint32(32 - d))
    ks = (k1, k2, k1 ^ k2 ^ np.uint32(0x1BD11BDA))
    rotations = ((13, 15, 26, 6), (17, 29, 16, 24))
    x1 = x1 + ks[0]
    x2 = x2 + ks[1]
    for i in range(5):
        for r in rotations[i % 2]:
            x1 = x1 + x2
            x2 = rotl(x2, r)
            x2 = x2 ^ x1
        x1 = x1 + ks[(i + 1) % 3]
        x2 = x2 + ks[(i + 2) % 3] + np.uint32(i + 1)
    return x1, x2


def _gumbel_tables():
    k1, k2 = np.uint32(0), np.uint32(42)
    gs = []
    for i in range(1, NL):
        f1, f2 = _threefry2x32(k1, k2, np.uint32(0), np.uint32(i))
        n = B * ST * T
        c = np.arange(n, dtype=np.uint32)
        o1, o2 = _threefry2x32(f1, f2, np.zeros(n, np.uint32), c)
        bits = o1 ^ o2
        u = ((bits >> np.uint32(9)) | np.uint32(0x3F800000)).view(np.float32)
        u = np.maximum(np.float32(0.0), u - np.float32(1.0))
        u = u.reshape(B * ST, T)
        gi = -np.log(-np.log(u + np.float32(1e-10)) + np.float32(1e-10))
        gs.append(gi.astype(np.float32).reshape(B, ST, T))
    return np.stack(gs, axis=1).reshape(B, NL - 1, ST * T)


_G_TABLE = _gumbel_tables()


def kernel(logits, global_feat, input_feat):
    w = jax.nn.softmax(logits, axis=1)
    cums = jnp.cumsum(w, axis=1)
    g = jnp.asarray(_G_TABLE)

    mc, idx = pl.pallas_call(
        _tc_body,
        grid=(B // BB,),
        in_specs=[
            pl.BlockSpec((BB, T), lambda i: (i, 0)),
            pl.BlockSpec((BB, T), lambda i: (i, 0)),
            pl.BlockSpec((BB, NL - 1, ST * T), lambda i: (i, 0, 0)),
            pl.BlockSpec((BB, T, D), lambda i: (i, 0, 0)),
            pl.BlockSpec((ST * T, ST * T), lambda i: (0, 0)),
            pl.BlockSpec((ST * T, T), lambda i: (0, 0)),
        ],
        out_specs=[
            pl.BlockSpec((BB, D), lambda i: (i, 0)),
            pl.BlockSpec((BB, NL), lambda i: (i, 0)),
        ],
        out_shape=[
            jax.ShapeDtypeStruct((B, D), jnp.float32),
            jax.ShapeDtypeStruct((B, NL), jnp.int32),
        ],
    )(w, cums, g, global_feat,
      jnp.asarray(_SEG_M), jnp.asarray(_FOLD_M))

    return _sc_gather(input_feat, idx.reshape(-1), mc)
